# baseline (speedup 1.0000x reference)
import functools

import jax
import jax.numpy as jnp
from jax import lax
from jax.experimental import pallas as pl
from jax.experimental.pallas import tpu as pltpu

_TOKENS = 4096


def _conv2d(x, w, b=None, stride=1, padding=0):
    out = lax.conv_general_dilated(
        x, w, (stride, stride), ((padding, padding), (padding, padding)),
        dimension_numbers=('NCHW', 'OIHW', 'NCHW'))
    if b is not None:
        out = out + b[None, :, None, None]
    return out


def _conv_transpose2d(x, w, b, stride, padding):
    k = w.shape[2]
    w_t = jnp.transpose(w[:, :, ::-1, ::-1], (1, 0, 2, 3))
    pad = k - 1 - padding
    out = lax.conv_general_dilated(
        x, w_t, (1, 1), ((pad, pad), (pad, pad)), lhs_dilation=(stride, stride),
        dimension_numbers=('NCHW', 'OIHW', 'NCHW'))
    return out + b[None, :, None, None]


def _res_stack(x, layers):
    for (w1, w2) in layers:
        y = jax.nn.relu(x)
        y = _conv2d(y, w1, None, 1, 1)
        y = jax.nn.relu(y)
        y = _conv2d(y, w2, None, 1, 0)
        x = x + y
    return jax.nn.relu(x)


def _vq_body(z_ref, embt_ref, e2_ref, idx_ref, md_ref, bmin_ref, barg_ref,
             *, d, C, nchunks):
    c = pl.program_id(1)
    z = z_ref[...]
    if d >= 16:
        s = jnp.dot(z, embt_ref[...], preferred_element_type=jnp.float32)
        s = e2_ref[...] - 2.0 * s
    else:
        embt = embt_ref[...]
        acc = z[:, 0:1] * embt[0:1, :]
        for k in range(1, d):
            acc = acc + z[:, k:k + 1] * embt[k:k + 1, :]
        s = e2_ref[...] - 2.0 * acc
    m = jnp.min(s, axis=1)
    ii = lax.broadcasted_iota(jnp.int32, s.shape, 1)
    a = jnp.min(jnp.where(s == m[:, None], ii, jnp.int32(2 ** 30)), axis=1)
    a = a + c * C

    @pl.when(c == 0)
    def _():
        bmin_ref[...] = m
        barg_ref[...] = a

    @pl.when(c > 0)
    def _():
        pm = bmin_ref[...]
        upd = m < pm
        bmin_ref[...] = jnp.where(upd, m, pm)
        barg_ref[...] = jnp.where(upd, a, barg_ref[...])

    @pl.when(c == nchunks - 1)
    def _():
        idx_ref[...] = barg_ref[...]
        md_ref[...] = bmin_ref[...] + jnp.sum(z * z, axis=1)


def _vq_argmin(flat, embt, e2):
    d, n = embt.shape
    T = 1024
    C = min(n, 1024)
    nchunks = n // C
    grid = (_TOKENS // T, nchunks)
    body = functools.partial(_vq_body, d=d, C=C, nchunks=nchunks)
    idx, md = pl.pallas_call(
        body,
        grid=grid,
        in_specs=[
            pl.BlockSpec((T, d), lambda t, c: (t, 0)),
            pl.BlockSpec((d, C), lambda t, c: (0, c)),
            pl.BlockSpec((1, C), lambda t, c: (0, c)),
        ],
        out_specs=[
            pl.BlockSpec((T,), lambda t, c: (t,)),
            pl.BlockSpec((T,), lambda t, c: (t,)),
        ],
        out_shape=[
            jax.ShapeDtypeStruct((_TOKENS,), jnp.int32),
            jax.ShapeDtypeStruct((_TOKENS,), jnp.float32),
        ],
        scratch_shapes=[
            pltpu.VMEM((T,), jnp.float32),
            pltpu.VMEM((T,), jnp.int32),
        ],
    )(flat, embt, e2)
    return idx, md


def kernel(x, params, gumbel_u):
    p = params
    h = _conv2d(x, p['enc_w1'], p['enc_b1'], 2, 1)
    h = jax.nn.relu(h)
    h = _conv2d(h, p['enc_w2'], p['enc_b2'], 2, 1)
    h = jax.nn.relu(h)
    h = _conv2d(h, p['enc_w3'], p['enc_b3'], 1, 1)
    h = _res_stack(h, [(p['enc_r1_w1'], p['enc_r1_w2']),
                       (p['enc_r2_w1'], p['enc_r2_w2'])])

    qf = h.reshape(-1, 128)
    N = qf.shape[0]
    qp = qf @ p['wq'].T + p['bq']
    kf = p['qkeys'].reshape(8, 128)
    kp = kf @ p['wk'].T + p['bk']
    qh = jnp.transpose(qp.reshape(N, 2, 64), (1, 0, 2))
    kh = jnp.transpose(kp.reshape(8, 2, 64), (1, 0, 2))
    scores = jnp.einsum('hqd,hkd->hqk', qh, kh) / jnp.sqrt(64.0)
    att = jnp.mean(jax.nn.softmax(scores, axis=-1), axis=0)[None]
    g = -jnp.log(-jnp.log(gumbel_u + 1e-20) + 1e-20)
    y_soft = jax.nn.softmax(att + g, axis=2)
    route = jnp.argmax(y_soft, axis=2)[0]

    cb_loss = 0.0
    zs = []
    for i in range(8):
        emb = p['vq_emb'][i]
        d = emb.shape[1]
        z = _conv2d(h, p['vq_mw'][i], p['vq_mb'][i], 1, 0)
        zi = jnp.transpose(z, (0, 2, 3, 1))
        flat = zi.reshape(-1, d)
        embt = emb.T
        e2 = jnp.sum(emb * emb, axis=1)[None, :]
        idx, md = _vq_argmin(flat, embt, e2)
        cb_loss = cb_loss + 1.25 * jnp.sum(md) / (_TOKENS * d)
        q = jnp.take(emb, idx, axis=0).reshape(zi.shape)
        q_nchw = jnp.transpose(q, (0, 3, 1, 2))
        nv = _conv2d(q_nchw, p['vq_cw'][i], p['vq_cb'][i], 1, 1)
        zs.append(nv.reshape(-1, 128))
    extra_loss = cb_loss / 8.0

    Zs = jnp.stack(zs, axis=1)
    state = jnp.take_along_axis(Zs, route[:, None, None], axis=1)[:, 0, :]
    state = state.reshape(64, 128, 8, 8)

    d_ = _res_stack(state, [(p['dec_r1_w1'], p['dec_r1_w2']),
                            (p['dec_r2_w1'], p['dec_r2_w2'])])
    d_ = _conv_transpose2d(d_, p['dec_t1_w'], p['dec_t1_b'], 2, 1)
    d_ = jax.nn.relu(d_)
    x_recon = _conv_transpose2d(d_, p['dec_t2_w'], p['dec_t2_b'], 2, 1)
    recon_loss = jnp.mean((x - x_recon) ** 2)
    loss = recon_loss + extra_loss
    return loss, x_recon
